# Initial kernel scaffold; baseline (speedup 1.0000x reference)
import jax
import jax.numpy as jnp
from jax.experimental import pallas as pl


def kernel(x, edge_index, edge_attr, global_features, v_re, v_im, params):
    raise NotImplementedError("write your pallas kernel here")


# speedup vs baseline: 2.2562x; 2.2562x over previous
import functools

import jax
import jax.numpy as jnp
from jax import lax
from jax.experimental import pallas as pl
from jax.experimental.pallas import tpu as pltpu
from jax.experimental.pallas import tpu_sc as plsc

L = 64
LA = 32
N = 10000
E = 160000
MP_STEPS = 6

NC = 2
NS = 16
NW = NC * NS
CH = 128
NCH = E // CH
TRIPS = (NCH + NW - 1) // NW
ZR = N // NS

_f32 = jnp.float32

_SC_PARAMS = pltpu.CompilerParams(use_tc_tiling_on_sc=False)


def _mesh():
    return plsc.VectorSubcoreMesh(core_axis_name="c", subcore_axis_name="s")


def _sc_gather_sum(p, q, row, col):
    @functools.partial(
        pl.kernel,
        out_type=jax.ShapeDtypeStruct((E, L), _f32),
        mesh=_mesh(),
        scratch_types=[
            pltpu.VMEM((CH,), jnp.int32),
            pltpu.VMEM((CH,), jnp.int32),
            pltpu.VMEM((CH, L), _f32),
            pltpu.VMEM((CH, L), _f32),
            pltpu.SemaphoreType.DMA,
            pltpu.SemaphoreType.DMA,
        ],
        compiler_params=_SC_PARAMS,
    )
    def k(p_hbm, q_hbm, row_hbm, col_hbm, out_hbm, ir_v, ic_v, ba_v, bb_v, sa, sb):
        w = lax.axis_index("s") * NC + lax.axis_index("c")

        def body(t, carry):
            c = w + t * NW

            @pl.when(c < NCH)
            def _():
                base = c * CH
                pltpu.sync_copy(row_hbm.at[pl.ds(base, CH)], ir_v)
                pltpu.sync_copy(col_hbm.at[pl.ds(base, CH)], ic_v)
                cpa = pltpu.async_copy(p_hbm.at[ir_v], ba_v, sa)
                cpb = pltpu.async_copy(q_hbm.at[ic_v], bb_v, sb)
                cpa.wait()
                cpb.wait()

                def add_row(i, carry2):
                    for j in range(L // 16):
                        s = pl.ds(j * 16, 16)
                        ba_v[i, s] = ba_v[i, s] + bb_v[i, s]
                    return carry2

                lax.fori_loop(0, CH, add_row, 0)
                pltpu.sync_copy(ba_v, out_hbm.at[pl.ds(base, CH)])

            return carry

        lax.fori_loop(0, TRIPS, body, 0)

    return k(p, q, row, col)


def _sc_scatter(msg, idx):
    @functools.partial(
        pl.kernel,
        out_type=jax.ShapeDtypeStruct((NC * N, L), _f32),
        mesh=_mesh(),
        scratch_types=[
            pltpu.VMEM((CH,), jnp.int32),
            pltpu.VMEM((CH, L), _f32),
            pltpu.VMEM_SHARED((N, L), _f32),
            pltpu.SemaphoreType.DMA,
        ],
        compiler_params=_SC_PARAMS,
    )
    def k(msg_hbm, idx_hbm, out_hbm, idx_v, buf_v, acc_sh, sem):
        cid = lax.axis_index("c")
        sid = lax.axis_index("s")
        w = sid * NC + cid

        def zrow(i, carry):
            for j in range(L // 16):
                buf_v[i, pl.ds(j * 16, 16)] = jnp.zeros((16,), _f32)
            return carry

        lax.fori_loop(0, CH, zrow, 0)
        for kk in range(5):
            pltpu.sync_copy(buf_v.at[pl.ds(0, 125)],
                            acc_sh.at[pl.ds(sid * ZR + kk * 125, 125)])
        plsc.subcore_barrier()

        def body(t, carry):
            c = w + t * NW

            @pl.when(c < NCH)
            def _():
                base = c * CH
                pltpu.sync_copy(idx_hbm.at[pl.ds(base, CH)], idx_v)
                pltpu.sync_copy(msg_hbm.at[pl.ds(base, CH)], buf_v)
                pltpu.sync_copy(buf_v, acc_sh.at[idx_v], add=True)

            return carry

        lax.fori_loop(0, TRIPS, body, 0)
        plsc.subcore_barrier()
        pltpu.sync_copy(acc_sh.at[pl.ds(sid * ZR, ZR)],
                        out_hbm.at[pl.ds(cid * N + sid * ZR, ZR)])

    return k(msg, idx)


def _sc_counts(idx):
    @functools.partial(
        pl.kernel,
        out_type=jax.ShapeDtypeStruct((NC * N, 16), _f32),
        mesh=_mesh(),
        scratch_types=[
            pltpu.VMEM((CH,), jnp.int32),
            pltpu.VMEM((CH, 16), _f32),
            pltpu.VMEM_SHARED((N, 16), _f32),
            pltpu.SemaphoreType.DMA,
        ],
        compiler_params=_SC_PARAMS,
    )
    def k(idx_hbm, out_hbm, idx_v, buf_v, acc_sh, sem):
        cid = lax.axis_index("c")
        sid = lax.axis_index("s")
        w = sid * NC + cid

        def zrow(i, carry):
            buf_v[i, :] = jnp.zeros((16,), _f32)
            return carry

        lax.fori_loop(0, CH, zrow, 0)
        for kk in range(5):
            pltpu.sync_copy(buf_v.at[pl.ds(0, 125)],
                            acc_sh.at[pl.ds(sid * ZR + kk * 125, 125)])
        plsc.subcore_barrier()

        def orow(i, carry):
            buf_v[i, :] = jnp.full((16,), 1.0, _f32)
            return carry

        lax.fori_loop(0, CH, orow, 0)

        def body(t, carry):
            c = w + t * NW

            @pl.when(c < NCH)
            def _():
                base = c * CH
                pltpu.sync_copy(idx_hbm.at[pl.ds(base, CH)], idx_v)
                pltpu.sync_copy(buf_v, acc_sh.at[idx_v], add=True)

            return carry

        lax.fori_loop(0, TRIPS, body, 0)
        plsc.subcore_barrier()
        pltpu.sync_copy(acc_sh.at[pl.ds(sid * ZR, ZR)],
                        out_hbm.at[pl.ds(cid * N + sid * ZR, ZR)])

    return k(idx)


def _sc_apply(hm, col, gate, row):
    @functools.partial(
        pl.kernel,
        out_type=jax.ShapeDtypeStruct((NC * N, LA), _f32),
        mesh=_mesh(),
        scratch_types=[
            pltpu.VMEM((CH,), jnp.int32),
            pltpu.VMEM((CH,), jnp.int32),
            pltpu.VMEM((CH, LA), _f32),
            pltpu.VMEM((CH, LA), _f32),
            pltpu.VMEM_SHARED((N, LA), _f32),
            pltpu.SemaphoreType.DMA,
        ],
        compiler_params=_SC_PARAMS,
    )
    def k(hm_hbm, col_hbm, gate_hbm, row_hbm, out_hbm,
          ir_v, ic_v, bh_v, bg_v, acc_sh, sem):
        cid = lax.axis_index("c")
        sid = lax.axis_index("s")
        w = sid * NC + cid

        def zrow(i, carry):
            for j in range(LA // 16):
                bh_v[i, pl.ds(j * 16, 16)] = jnp.zeros((16,), _f32)
            return carry

        lax.fori_loop(0, CH, zrow, 0)
        for kk in range(5):
            pltpu.sync_copy(bh_v.at[pl.ds(0, 125)],
                            acc_sh.at[pl.ds(sid * ZR + kk * 125, 125)])
        plsc.subcore_barrier()

        def body(t, carry):
            c = w + t * NW

            @pl.when(c < NCH)
            def _():
                base = c * CH
                pltpu.sync_copy(col_hbm.at[pl.ds(base, CH)], ic_v)
                pltpu.sync_copy(row_hbm.at[pl.ds(base, CH)], ir_v)
                pltpu.async_copy(hm_hbm.at[ic_v], bh_v, sem).wait()
                pltpu.sync_copy(gate_hbm.at[pl.ds(base, CH)], bg_v)

                def mrow(i, carry2):
                    for j in range(LA // 16):
                        s = pl.ds(j * 16, 16)
                        bh_v[i, s] = bh_v[i, s] * bg_v[i, s]
                    return carry2

                lax.fori_loop(0, CH, mrow, 0)
                pltpu.sync_copy(bh_v, acc_sh.at[ir_v], add=True)

            return carry

        lax.fori_loop(0, TRIPS, body, 0)
        plsc.subcore_barrier()
        pltpu.sync_copy(acc_sh.at[pl.ds(sid * ZR, ZR)],
                        out_hbm.at[pl.ds(cid * N + sid * ZR, ZR)])

    return k(hm, col, gate, row)


_EB = 2000
_NB = 2000

def _dot(a, b):
    return jnp.dot(a, b, preferred_element_type=_f32)


def _full(shape):
    return pl.BlockSpec(shape, lambda i: (0,) * len(shape))


def _node_enc(x, w0, b0, w1, b1, w1a, w1b):
    def body(x_r, w0_r, b0_r, w1_r, b1_r, wa_r, wb_r, nl_o, p_o, q_o):
        h = jnp.maximum(_dot(x_r[...], w0_r[...]) + b0_r[...], 0.0)
        nl = _dot(h, w1_r[...]) + b1_r[...]
        nl_o[...] = nl
        p_o[...] = _dot(nl, wa_r[...])
        q_o[...] = _dot(nl, wb_r[...])

    g = N // _NB
    return pl.pallas_call(
        body,
        grid=(g,),
        in_specs=[
            pl.BlockSpec((_NB, 9), lambda i: (i, 0)),
            _full((9, L)), _full((1, L)), _full((L, L)), _full((1, L)),
            _full((L, L)), _full((L, L)),
        ],
        out_specs=[
            pl.BlockSpec((_NB, L), lambda i: (i, 0)),
            pl.BlockSpec((_NB, L), lambda i: (i, 0)),
            pl.BlockSpec((_NB, L), lambda i: (i, 0)),
        ],
        out_shape=[jax.ShapeDtypeStruct((N, L), _f32)] * 3,
    )(x, w0, b0, w1, b1, w1a, w1b)


def _edge_enc(ea, w0, b0, w1, b1):
    def body(e_r, w0_r, b0_r, w1_r, b1_r, o_r):
        h = jnp.maximum(_dot(e_r[...], w0_r[...]) + b0_r[...], 0.0)
        o_r[...] = _dot(h, w1_r[...]) + b1_r[...]

    g = E // _EB
    return pl.pallas_call(
        body,
        grid=(g,),
        in_specs=[
            pl.BlockSpec((_EB, 8), lambda i: (i, 0)),
            _full((8, L)), _full((1, L)), _full((L, L)), _full((1, L)),
        ],
        out_specs=pl.BlockSpec((_EB, L), lambda i: (i, 0)),
        out_shape=jax.ShapeDtypeStruct((E, L), _f32),
    )(ea, w0, b0, w1, b1)


def _global_enc(gf, w0, b0, w1, b1):
    def body(g_r, w0_r, b0_r, w1_r, b1_r, o_r):
        h = jnp.maximum(_dot(g_r[...], w0_r[...]) + b0_r[...], 0.0)
        o_r[...] = _dot(h, w1_r[...]) + b1_r[...]

    return pl.pallas_call(
        body,
        grid=(1,),
        in_specs=[_full((1, 4)), _full((4, L)), _full((1, L)),
                  _full((L, L)), _full((1, L))],
        out_specs=_full((1, L)),
        out_shape=jax.ShapeDtypeStruct((1, L), _f32),
    )(gf, w0, b0, w1, b1)


def _edge_step(el, es, s_gath, g, wske, wsks, bsk, w1c, w1d, b1, w2, b2,
               gate_w=None, gate_b=None):
    with_gate = gate_w is not None

    def body(*refs):
        if with_gate:
            (el_r, es_r, sg_r, g_r, wske_r, wsks_r, bsk_r, w1c_r, w1d_r,
             b1_r, w2_r, b2_r, wg_r, bg_r, ne_o, esum_o, gate_o) = refs
        else:
            (el_r, es_r, sg_r, g_r, wske_r, wsks_r, bsk_r, w1c_r, w1d_r,
             b1_r, w2_r, b2_r, ne_o, esum_o) = refs
        ei = _dot(el_r[...], wske_r[...]) + _dot(es_r[...], wsks_r[...]) + bsk_r[...]
        pre = sg_r[...] + _dot(ei, w1c_r[...]) + _dot(g_r[...], w1d_r[...]) + b1_r[...]
        ne = ei + _dot(jnp.maximum(pre, 0.0), w2_r[...]) + b2_r[...]
        ne_o[...] = ne

        @pl.when(pl.program_id(0) == 0)
        def _():
            esum_o[...] = jnp.zeros_like(esum_o)

        esum_o[...] += jnp.sum(ne, axis=0, keepdims=True)
        if with_gate:
            gate_o[...] = jax.nn.sigmoid(_dot(ne, wg_r[...]) + bg_r[...])

    g_steps = E // _EB
    in_specs = [
        pl.BlockSpec((_EB, L), lambda i: (i, 0)),
        pl.BlockSpec((_EB, L), lambda i: (i, 0)),
        pl.BlockSpec((_EB, L), lambda i: (i, 0)),
        _full((1, L)), _full((L, L)), _full((L, L)), _full((1, L)),
        _full((L, L)), _full((L, L)), _full((1, L)),
        _full((L, L)), _full((1, L)),
    ]
    out_specs = [
        pl.BlockSpec((_EB, L), lambda i: (i, 0)),
        pl.BlockSpec((1, L), lambda i: (0, 0)),
    ]
    out_shape = [jax.ShapeDtypeStruct((E, L), _f32),
                 jax.ShapeDtypeStruct((1, L), _f32)]
    args = [el, es, s_gath, g, wske, wsks, bsk, w1c, w1d, b1, w2, b2]
    if with_gate:
        in_specs += [_full((L, LA)), _full((1, LA))]
        out_specs.append(pl.BlockSpec((_EB, LA), lambda i: (i, 0)))
        out_shape.append(jax.ShapeDtypeStruct((E, LA), _f32))
        args += [gate_w, gate_b]
    return pl.pallas_call(
        body, grid=(g_steps,), in_specs=in_specs, out_specs=out_specs,
        out_shape=out_shape,
    )(*args)


def _node_step(nd, parts, cnts, g, w1a, w1b, w1c, b1, w2, b2, wx, wy,
               proj_bias=None):
    with_bias = proj_bias is not None
    xw = LA if with_bias else L

    def body(*refs):
        if with_bias:
            (nd_r, a0_r, a1_r, c0_r, c1_r, g_r, w1a_r, w1b_r, w1c_r, b1_r,
             w2_r, b2_r, wx_r, wy_r, bp_r, nn_o, nsum_o, x_o, y_o) = refs
        else:
            (nd_r, a0_r, a1_r, c0_r, c1_r, g_r, w1a_r, w1b_r, w1c_r, b1_r,
             w2_r, b2_r, wx_r, wy_r, nn_o, nsum_o, x_o, y_o) = refs
        cnt = c0_r[...][:, :1] + c1_r[...][:, :1]
        scale = 1.0 / jnp.maximum(cnt, 1.0)
        agg = (a0_r[...] + a1_r[...]) * scale
        pre = (_dot(nd_r[...], w1a_r[...]) + _dot(agg, w1b_r[...])
               + _dot(g_r[...], w1c_r[...]) + b1_r[...])
        nn = nd_r[...] + _dot(jnp.maximum(pre, 0.0), w2_r[...]) + b2_r[...]
        nn_o[...] = nn

        @pl.when(pl.program_id(0) == 0)
        def _():
            nsum_o[...] = jnp.zeros_like(nsum_o)

        nsum_o[...] += jnp.sum(nn, axis=0, keepdims=True)
        if with_bias:
            x_o[...] = _dot(nn, wx_r[...]) + bp_r[...]
        else:
            x_o[...] = _dot(nn, wx_r[...])
        y_o[...] = _dot(nn, wy_r[...])

    g_steps = N // _NB
    off = N // _NB
    in_specs = [
        pl.BlockSpec((_NB, L), lambda i: (i, 0)),
        pl.BlockSpec((_NB, L), lambda i: (i, 0)),
        pl.BlockSpec((_NB, L), lambda i, o=off: (i + o, 0)),
        pl.BlockSpec((_NB, 16), lambda i: (i, 0)),
        pl.BlockSpec((_NB, 16), lambda i, o=off: (i + o, 0)),
        _full((1, L)), _full((L, L)), _full((L, L)), _full((L, L)),
        _full((1, L)), _full((L, L)), _full((1, L)),
        _full((L, xw)), _full((L, L)),
    ]
    args = [nd, parts, parts, cnts, cnts, g, w1a, w1b, w1c, b1, w2, b2, wx, wy]
    if with_bias:
        in_specs.append(_full((1, LA)))
        args.append(proj_bias)
    return pl.pallas_call(
        body,
        grid=(g_steps,),
        in_specs=in_specs,
        out_specs=[
            pl.BlockSpec((_NB, L), lambda i: (i, 0)),
            pl.BlockSpec((1, L), lambda i: (0, 0)),
            pl.BlockSpec((_NB, xw), lambda i: (i, 0)),
            pl.BlockSpec((_NB, L), lambda i: (i, 0)),
        ],
        out_shape=[
            jax.ShapeDtypeStruct((N, L), _f32),
            jax.ShapeDtypeStruct((1, L), _f32),
            jax.ShapeDtypeStruct((N, xw), _f32),
            jax.ShapeDtypeStruct((N, L), _f32),
        ],
    )(*args)


def _global_step(g, nsum, esum, wga, wgb, wgc, b1, w2, b2):
    def body(g_r, ns_r, es_r, wga_r, wgb_r, wgc_r, b1_r, w2_r, b2_r, o_r):
        nmean = ns_r[...] * (1.0 / N)
        emean = es_r[...] * (1.0 / E)
        pre = (_dot(g_r[...], wga_r[...]) + _dot(nmean, wgb_r[...])
               + _dot(emean, wgc_r[...]) + b1_r[...])
        o_r[...] = g_r[...] + _dot(jnp.maximum(pre, 0.0), w2_r[...]) + b2_r[...]

    return pl.pallas_call(
        body,
        grid=(1,),
        in_specs=[_full((1, L))] * 3 + [_full((L, L))] * 3
        + [_full((1, L)), _full((L, L)), _full((1, L))],
        out_specs=_full((1, L)),
        out_shape=jax.ShapeDtypeStruct((1, L), _f32),
    )(g, nsum, esum, wga, wgb, wgc, b1, w2, b2)


def _apply_input(cache, v2, wa, wv, b1, w2, b2, wm, bm):
    def body(c_r, v_r, wa_r, wv_r, b1_r, w2_r, b2_r, wm_r, bm_r, h_o, hm_o):
        pre = _dot(c_r[...], wa_r[...]) + _dot(v_r[...], wv_r[...]) + b1_r[...]
        h = jnp.maximum(pre, 0.0)
        h = _dot(h, w2_r[...]) + b2_r[...]
        h_o[...] = h
        hm_o[...] = _dot(h, wm_r[...]) + bm_r[...]

    g_steps = N // _NB
    return pl.pallas_call(
        body,
        grid=(g_steps,),
        in_specs=[
            pl.BlockSpec((_NB, LA), lambda i: (i, 0)),
            pl.BlockSpec((_NB, 2), lambda i: (i, 0)),
            _full((LA, LA)), _full((2, LA)), _full((1, LA)),
            _full((LA, LA)), _full((1, LA)),
            _full((LA, LA)), _full((1, LA)),
        ],
        out_specs=[
            pl.BlockSpec((_NB, LA), lambda i: (i, 0)),
            pl.BlockSpec((_NB, LA), lambda i: (i, 0)),
        ],
        out_shape=[jax.ShapeDtypeStruct((N, LA), _f32)] * 2,
    )(cache, v2, wa, wv, b1, w2, b2, wm, bm)


def _apply_final(h, parts, cnts, v2, wha, whb, b1, w2, b2, wo, bo):
    def body(h_r, a0_r, a1_r, c0_r, c1_r, v_r, wha_r, whb_r, b1_r, w2_r,
             b2_r, wo_r, bo_r, o_r):
        cnt = c0_r[...][:, :1] + c1_r[...][:, :1]
        scale = 1.0 / jnp.maximum(cnt, 1.0)
        agg = (a0_r[...] + a1_r[...]) * scale
        pre = _dot(h_r[...], wha_r[...]) + _dot(agg, whb_r[...]) + b1_r[...]
        h2 = h_r[...] + _dot(jnp.maximum(pre, 0.0), w2_r[...]) + b2_r[...]
        o_r[...] = v_r[...] + _dot(h2, wo_r[...]) + bo_r[...]

    g_steps = N // _NB
    off = N // _NB
    return pl.pallas_call(
        body,
        grid=(g_steps,),
        in_specs=[
            pl.BlockSpec((_NB, LA), lambda i: (i, 0)),
            pl.BlockSpec((_NB, LA), lambda i: (i, 0)),
            pl.BlockSpec((_NB, LA), lambda i, o=off: (i + o, 0)),
            pl.BlockSpec((_NB, 16), lambda i: (i, 0)),
            pl.BlockSpec((_NB, 16), lambda i, o=off: (i + o, 0)),
            pl.BlockSpec((_NB, 2), lambda i: (i, 0)),
            _full((LA, LA)), _full((LA, LA)), _full((1, LA)),
            _full((LA, LA)), _full((1, LA)),
            _full((LA, 2)), _full((1, 2)),
        ],
        out_specs=pl.BlockSpec((_NB, 2), lambda i: (i, 0)),
        out_shape=jax.ShapeDtypeStruct((N, 2), _f32),
    )(h, parts, parts, cnts, cnts, v2, wha, whb, b1, w2, b2, wo, bo)


def _r(b):
    return b.reshape(1, -1)


def _forward(x, edge_index, edge_attr, global_features, v_re, v_im, params):
    prm = params
    row = edge_index[0]
    col = edge_index[1]

    proc = prm['processor']
    skips = prm['edge_skip']
    e1 = [p['edge_mlp'][0][0] for p in proc]
    w1a = [w[0:L] for w in e1]
    w1b = [w[L:2 * L] for w in e1]
    w1c = [w[2 * L:3 * L] for w in e1]
    w1d = [w[3 * L:4 * L] for w in e1]

    (ne_w0, ne_b0), (ne_w1, ne_b1) = prm['node_enc']
    (ee_w0, ee_b0), (ee_w1, ee_b1) = prm['edge_enc']
    (ge_w0, ge_b0), (ge_w1, ge_b1) = prm['global_enc']

    node, p_cur, q_cur = _node_enc(x, ne_w0, _r(ne_b0), ne_w1, _r(ne_b1),
                                   w1a[0], w1b[0])
    e_saved = _edge_enc(edge_attr, ee_w0, _r(ee_b0), ee_w1, _r(ee_b1))
    g = _global_enc(global_features, ge_w0, _r(ge_b0), ge_w1, _r(ge_b1))

    cnts = _sc_counts(row)

    edge = e_saved
    gate = None
    cache = None
    (gate_w, gate_b) = prm['edge_gate']
    (proj_w, proj_b) = prm['node_proj']
    aux = {}
    for i in range(MP_STEPS):
        sk_w, sk_b = skips[i]
        pe = proc[i]['edge_mlp']
        pn = proc[i]['node_mlp']
        pg = proc[i]['global_mlp']
        b1 = pe[0][1]
        w2, b2 = pe[1]
        sgath = _sc_gather_sum(p_cur, q_cur, row, col)
        last = i == MP_STEPS - 1
        eres = _edge_step(edge, e_saved, sgath, g,
                          sk_w[0:L], sk_w[L:2 * L], _r(sk_b),
                          w1c[i], w1d[i], _r(b1), w2, _r(b2),
                          gate_w=gate_w if last else None,
                          gate_b=_r(gate_b) if last else None)
        if last:
            edge, esum, gate = eres
        else:
            edge, esum = eres
        parts = _sc_scatter(edge, row)
        n1 = pn[0][0]
        if last:
            node, nsum, cache, _dummy = _node_step(
                node, parts, cnts, g,
                n1[0:L], n1[L:2 * L], n1[2 * L:3 * L], _r(pn[0][1]),
                pn[1][0], _r(pn[1][1]), proj_w, pn[1][0],
                proj_bias=_r(proj_b))
        else:
            node, nsum, p_cur, q_cur = _node_step(
                node, parts, cnts, g,
                n1[0:L], n1[L:2 * L], n1[2 * L:3 * L], _r(pn[0][1]),
                pn[1][0], _r(pn[1][1]), w1a[i + 1], w1b[i + 1])
        g1 = pg[0][0]
        g = _global_step(g, nsum, esum,
                         g1[0:L], g1[L:2 * L], g1[2 * L:3 * L],
                         _r(pg[0][1]), pg[1][0], _r(pg[1][1]))

    aux['node'] = node
    aux['edge'] = edge
    aux['g'] = g
    aux['gate'] = gate
    aux['cache'] = cache

    v2 = jnp.stack([v_re, v_im], axis=1)
    ai = prm['apply_input']
    a1w = ai[0][0]
    mp0 = prm['apply_mp'][0]
    (wm, bm) = mp0['msg_proj']
    h, hm = _apply_input(cache, v2, a1w[0:LA], a1w[LA:LA + 2], _r(ai[0][1]),
                         ai[1][0], _r(ai[1][1]), wm, _r(bm))
    aux['h'] = h
    aux['hm'] = hm

    aparts = _sc_apply(hm, col, gate, row)
    aux['aparts'] = aparts

    u1 = mp0['upd'][0][0]
    (wo, bo) = prm['apply_output']
    out2 = _apply_final(h, aparts, cnts, v2,
                        u1[0:LA], u1[LA:2 * LA], _r(mp0['upd'][0][1]),
                        mp0['upd'][1][0], _r(mp0['upd'][1][1]), wo, _r(bo))
    aux['out2'] = out2
    return out2[:, 0], out2[:, 1], aux


def kernel(x, edge_index, edge_attr, global_features, v_re, v_im, params):
    out_re, out_im, _ = _forward(x, edge_index, edge_attr, global_features,
                                 v_re, v_im, params)
    return out_re, out_im


# speedup vs baseline: 2.3144x; 1.0258x over previous
import functools

import jax
import jax.numpy as jnp
from jax import lax
from jax.experimental import pallas as pl
from jax.experimental.pallas import tpu as pltpu
from jax.experimental.pallas import tpu_sc as plsc

L = 64
LA = 32
N = 10000
E = 160000
MP_STEPS = 6

NC = 2
NS = 16
NW = NC * NS
CH = 128
NCH = E // CH
NCHP = 1280
TRIPS = NCHP // NW
EP = NCHP * CH
ZR = N // NS

_f32 = jnp.float32

_SC_PARAMS = pltpu.CompilerParams(use_tc_tiling_on_sc=False)


def _mesh():
    return plsc.VectorSubcoreMesh(core_axis_name="c", subcore_axis_name="s")


def _sc_gather_sum(p, q, row2, col2):
    @functools.partial(
        pl.kernel,
        out_type=jax.ShapeDtypeStruct((EP, L), _f32),
        mesh=_mesh(),
        scratch_types=[
            pltpu.VMEM((TRIPS, CH), jnp.int32),
            pltpu.VMEM((TRIPS, CH), jnp.int32),
            pltpu.VMEM((2, CH, L), _f32),
            pltpu.VMEM((2, CH, L), _f32),
            pltpu.VMEM((2, CH, L), _f32),
            pltpu.SemaphoreType.DMA,
            pltpu.SemaphoreType.DMA,
            pltpu.SemaphoreType.DMA,
            pltpu.SemaphoreType.DMA,
            pltpu.SemaphoreType.DMA,
            pltpu.SemaphoreType.DMA,
        ],
        compiler_params=_SC_PARAMS,
    )
    def k(p_hbm, q_hbm, row_hbm, col_hbm, out_hbm, ir_v, ic_v, ba, bb, bo,
          sa0, sa1, sb0, sb1, so0, so1):
        w = lax.axis_index("s") * NC + lax.axis_index("c")
        start = w * TRIPS
        sa = [sa0, sa1]
        sb = [sb0, sb1]
        so = [so0, so1]
        pltpu.sync_copy(row_hbm.at[pl.ds(start, TRIPS), :], ir_v)
        pltpu.sync_copy(col_hbm.at[pl.ds(start, TRIPS), :], ic_v)
        for b in range(2):
            pltpu.async_copy(p_hbm.at[ir_v.at[b]], ba.at[b], sa[b])
            pltpu.async_copy(q_hbm.at[ic_v.at[b]], bb.at[b], sb[b])

        def body(u, carry):
            for b in range(2):
                t = 2 * u + b
                pltpu.make_async_copy(p_hbm.at[ir_v.at[t]], ba.at[b],
                                      sa[b]).wait()
                pltpu.make_async_copy(q_hbm.at[ic_v.at[t]], bb.at[b],
                                      sb[b]).wait()

                @pl.when(t >= 2)
                def _():
                    pltpu.make_async_copy(bo.at[b], out_hbm.at[pl.ds(0, CH)],
                                          so[b]).wait()

                def add_row(i, c2):
                    for j in range(L // 16):
                        s = pl.ds(j * 16, 16)
                        bo[b, i, s] = ba[b, i, s] + bb[b, i, s]
                    return c2

                lax.fori_loop(0, CH, add_row, 0)

                @pl.when(t + 2 < TRIPS)
                def _():
                    pltpu.async_copy(p_hbm.at[ir_v.at[t + 2]], ba.at[b], sa[b])
                    pltpu.async_copy(q_hbm.at[ic_v.at[t + 2]], bb.at[b], sb[b])

                pltpu.async_copy(bo.at[b],
                                 out_hbm.at[pl.ds((start + t) * CH, CH)], so[b])
            return carry

        lax.fori_loop(0, TRIPS // 2, body, 0)
        for b in range(2):
            pltpu.make_async_copy(bo.at[b], out_hbm.at[pl.ds(0, CH)],
                                  so[b]).wait()

    return k(p, q, row2, col2)


def _sc_scatter(msg, idx2):
    @functools.partial(
        pl.kernel,
        out_type=jax.ShapeDtypeStruct((NC * N, L), _f32),
        mesh=_mesh(),
        scratch_types=[
            pltpu.VMEM((TRIPS, CH), jnp.int32),
            pltpu.VMEM((2, CH, L), _f32),
            pltpu.VMEM_SHARED((N, L), _f32),
            pltpu.SemaphoreType.DMA,
            pltpu.SemaphoreType.DMA,
        ],
        compiler_params=_SC_PARAMS,
    )
    def k(msg_hbm, idx_hbm, out_hbm, idx_v, mb, acc_sh, sm0, sm1):
        cid = lax.axis_index("c")
        sid = lax.axis_index("s")
        w = sid * NC + cid
        start = w * TRIPS
        sm = [sm0, sm1]

        def zrow(i, carry):
            for j in range(L // 16):
                mb[0, i, pl.ds(j * 16, 16)] = jnp.zeros((16,), _f32)
            return carry

        lax.fori_loop(0, CH, zrow, 0)
        for kk in range(5):
            pltpu.sync_copy(mb.at[0, pl.ds(0, 125)],
                            acc_sh.at[pl.ds(sid * ZR + kk * 125, 125)])
        plsc.subcore_barrier()

        pltpu.sync_copy(idx_hbm.at[pl.ds(start, TRIPS), :], idx_v)
        for b in range(2):
            pltpu.async_copy(msg_hbm.at[pl.ds((start + b) * CH, CH)],
                             mb.at[b], sm[b])

        def body(u, carry):
            for b in range(2):
                t = 2 * u + b
                pltpu.make_async_copy(msg_hbm.at[pl.ds(0, CH)], mb.at[b],
                                      sm[b]).wait()

                @pl.when(start + t < NCH)
                def _():
                    pltpu.sync_copy(mb.at[b], acc_sh.at[idx_v.at[t]], add=True)

                @pl.when(t + 2 < TRIPS)
                def _():
                    pltpu.async_copy(
                        msg_hbm.at[pl.ds((start + t + 2) * CH, CH)],
                        mb.at[b], sm[b])
            return carry

        lax.fori_loop(0, TRIPS // 2, body, 0)
        plsc.subcore_barrier()
        pltpu.sync_copy(acc_sh.at[pl.ds(sid * ZR, ZR)],
                        out_hbm.at[pl.ds(cid * N + sid * ZR, ZR)])

    return k(msg, idx2)


def _sc_counts(idx2):
    @functools.partial(
        pl.kernel,
        out_type=jax.ShapeDtypeStruct((NC * N, 16), _f32),
        mesh=_mesh(),
        scratch_types=[
            pltpu.VMEM((TRIPS, CH), jnp.int32),
            pltpu.VMEM((CH, 16), _f32),
            pltpu.VMEM_SHARED((N, 16), _f32),
            pltpu.SemaphoreType.DMA,
        ],
        compiler_params=_SC_PARAMS,
    )
    def k(idx_hbm, out_hbm, idx_v, buf_v, acc_sh, sem):
        cid = lax.axis_index("c")
        sid = lax.axis_index("s")
        w = sid * NC + cid
        start = w * TRIPS

        def zrow(i, carry):
            buf_v[i, :] = jnp.zeros((16,), _f32)
            return carry

        lax.fori_loop(0, CH, zrow, 0)
        for kk in range(5):
            pltpu.sync_copy(buf_v.at[pl.ds(0, 125)],
                            acc_sh.at[pl.ds(sid * ZR + kk * 125, 125)])
        plsc.subcore_barrier()

        def orow(i, carry):
            buf_v[i, :] = jnp.full((16,), 1.0, _f32)
            return carry

        lax.fori_loop(0, CH, orow, 0)
        pltpu.sync_copy(idx_hbm.at[pl.ds(start, TRIPS), :], idx_v)

        def body(t, carry):
            @pl.when(start + t < NCH)
            def _():
                pltpu.sync_copy(buf_v, acc_sh.at[idx_v.at[t]], add=True)
            return carry

        lax.fori_loop(0, TRIPS, body, 0)
        plsc.subcore_barrier()
        pltpu.sync_copy(acc_sh.at[pl.ds(sid * ZR, ZR)],
                        out_hbm.at[pl.ds(cid * N + sid * ZR, ZR)])

    return k(idx2)


def _sc_apply(hm, col2, gate, row2):
    @functools.partial(
        pl.kernel,
        out_type=jax.ShapeDtypeStruct((NC * N, LA), _f32),
        mesh=_mesh(),
        scratch_types=[
            pltpu.VMEM((TRIPS, CH), jnp.int32),
            pltpu.VMEM((TRIPS, CH), jnp.int32),
            pltpu.VMEM((2, CH, LA), _f32),
            pltpu.VMEM((2, CH, LA), _f32),
            pltpu.VMEM_SHARED((N, LA), _f32),
            pltpu.SemaphoreType.DMA,
            pltpu.SemaphoreType.DMA,
            pltpu.SemaphoreType.DMA,
            pltpu.SemaphoreType.DMA,
        ],
        compiler_params=_SC_PARAMS,
    )
    def k(hm_hbm, col_hbm, gate_hbm, row_hbm, out_hbm,
          ir_v, ic_v, bh, bg, acc_sh, sh0, sh1, sg0, sg1):
        cid = lax.axis_index("c")
        sid = lax.axis_index("s")
        w = sid * NC + cid
        start = w * TRIPS
        sh = [sh0, sh1]
        sg = [sg0, sg1]

        def zrow(i, carry):
            for j in range(LA // 16):
                bh[0, i, pl.ds(j * 16, 16)] = jnp.zeros((16,), _f32)
            return carry

        lax.fori_loop(0, CH, zrow, 0)
        for kk in range(5):
            pltpu.sync_copy(bh.at[0, pl.ds(0, 125)],
                            acc_sh.at[pl.ds(sid * ZR + kk * 125, 125)])
        plsc.subcore_barrier()

        pltpu.sync_copy(col_hbm.at[pl.ds(start, TRIPS), :], ic_v)
        pltpu.sync_copy(row_hbm.at[pl.ds(start, TRIPS), :], ir_v)
        for b in range(2):
            pltpu.async_copy(hm_hbm.at[ic_v.at[b]], bh.at[b], sh[b])
            pltpu.async_copy(gate_hbm.at[pl.ds((start + b) * CH, CH)],
                             bg.at[b], sg[b])

        def body(u, carry):
            for b in range(2):
                t = 2 * u + b
                pltpu.make_async_copy(hm_hbm.at[ic_v.at[t]], bh.at[b],
                                      sh[b]).wait()
                pltpu.make_async_copy(gate_hbm.at[pl.ds(0, CH)], bg.at[b],
                                      sg[b]).wait()

                def mrow(i, c2):
                    for j in range(LA // 16):
                        s = pl.ds(j * 16, 16)
                        bh[b, i, s] = bh[b, i, s] * bg[b, i, s]
                    return c2

                lax.fori_loop(0, CH, mrow, 0)

                @pl.when(start + t < NCH)
                def _():
                    pltpu.sync_copy(bh.at[b], acc_sh.at[ir_v.at[t]], add=True)

                @pl.when(t + 2 < TRIPS)
                def _():
                    pltpu.async_copy(hm_hbm.at[ic_v.at[t + 2]], bh.at[b], sh[b])
                    pltpu.async_copy(
                        gate_hbm.at[pl.ds((start + t + 2) * CH, CH)],
                        bg.at[b], sg[b])
            return carry

        lax.fori_loop(0, TRIPS // 2, body, 0)
        plsc.subcore_barrier()
        pltpu.sync_copy(acc_sh.at[pl.ds(sid * ZR, ZR)],
                        out_hbm.at[pl.ds(cid * N + sid * ZR, ZR)])

    return k(hm, col2, gate, row2)


_EB = 2000
_NB = 2000

def _dot(a, b):
    return jnp.dot(a, b, preferred_element_type=_f32)


def _full(shape):
    return pl.BlockSpec(shape, lambda i: (0,) * len(shape))


def _node_enc(x, w0, b0, w1, b1, w1a, w1b):
    def body(x_r, w0_r, b0_r, w1_r, b1_r, wa_r, wb_r, nl_o, p_o, q_o):
        h = jnp.maximum(_dot(x_r[...], w0_r[...]) + b0_r[...], 0.0)
        nl = _dot(h, w1_r[...]) + b1_r[...]
        nl_o[...] = nl
        p_o[...] = _dot(nl, wa_r[...])
        q_o[...] = _dot(nl, wb_r[...])

    g = N // _NB
    return pl.pallas_call(
        body,
        grid=(g,),
        in_specs=[
            pl.BlockSpec((_NB, 9), lambda i: (i, 0)),
            _full((9, L)), _full((1, L)), _full((L, L)), _full((1, L)),
            _full((L, L)), _full((L, L)),
        ],
        out_specs=[
            pl.BlockSpec((_NB, L), lambda i: (i, 0)),
            pl.BlockSpec((_NB, L), lambda i: (i, 0)),
            pl.BlockSpec((_NB, L), lambda i: (i, 0)),
        ],
        out_shape=[jax.ShapeDtypeStruct((N, L), _f32)] * 3,
    )(x, w0, b0, w1, b1, w1a, w1b)


def _edge_enc(ea, w0, b0, w1, b1):
    def body(e_r, w0_r, b0_r, w1_r, b1_r, o_r):
        h = jnp.maximum(_dot(e_r[...], w0_r[...]) + b0_r[...], 0.0)
        o_r[...] = _dot(h, w1_r[...]) + b1_r[...]

    g = E // _EB
    return pl.pallas_call(
        body,
        grid=(g,),
        in_specs=[
            pl.BlockSpec((_EB, 8), lambda i: (i, 0)),
            _full((8, L)), _full((1, L)), _full((L, L)), _full((1, L)),
        ],
        out_specs=pl.BlockSpec((_EB, L), lambda i: (i, 0)),
        out_shape=jax.ShapeDtypeStruct((E, L), _f32),
    )(ea, w0, b0, w1, b1)


def _global_enc(gf, w0, b0, w1, b1):
    def body(g_r, w0_r, b0_r, w1_r, b1_r, o_r):
        h = jnp.maximum(_dot(g_r[...], w0_r[...]) + b0_r[...], 0.0)
        o_r[...] = _dot(h, w1_r[...]) + b1_r[...]

    return pl.pallas_call(
        body,
        grid=(1,),
        in_specs=[_full((1, 4)), _full((4, L)), _full((1, L)),
                  _full((L, L)), _full((1, L))],
        out_specs=_full((1, L)),
        out_shape=jax.ShapeDtypeStruct((1, L), _f32),
    )(gf, w0, b0, w1, b1)


def _edge_step(el, es, s_gath, g, wske, wsks, bsk, w1c, w1d, b1, w2, b2,
               gate_w=None, gate_b=None):
    with_gate = gate_w is not None

    def body(*refs):
        if with_gate:
            (el_r, es_r, sg_r, g_r, wske_r, wsks_r, bsk_r, w1c_r, w1d_r,
             b1_r, w2_r, b2_r, wg_r, bg_r, ne_o, esum_o, gate_o) = refs
        else:
            (el_r, es_r, sg_r, g_r, wske_r, wsks_r, bsk_r, w1c_r, w1d_r,
             b1_r, w2_r, b2_r, ne_o, esum_o) = refs
        ei = (_dot(el_r[...], wske_r[...]) + _dot(es_r[...], wsks_r[...])
              + bsk_r[...])
        pre = (sg_r[...] + _dot(ei, w1c_r[...]) + _dot(g_r[...], w1d_r[...])
               + b1_r[...])
        ne = ei + _dot(jnp.maximum(pre, 0.0), w2_r[...]) + b2_r[...]
        ne_o[...] = ne

        @pl.when(pl.program_id(0) == 0)
        def _():
            esum_o[...] = jnp.zeros_like(esum_o)

        esum_o[...] += jnp.sum(ne, axis=0, keepdims=True)
        if with_gate:
            gate_o[...] = jax.nn.sigmoid(_dot(ne, wg_r[...]) + bg_r[...])

    g_steps = E // _EB
    in_specs = [
        pl.BlockSpec((_EB, L), lambda i: (i, 0)),
        pl.BlockSpec((_EB, L), lambda i: (i, 0)),
        pl.BlockSpec((_EB, L), lambda i: (i, 0)),
        _full((1, L)), _full((L, L)), _full((L, L)), _full((1, L)),
        _full((L, L)), _full((L, L)), _full((1, L)),
        _full((L, L)), _full((1, L)),
    ]
    out_specs = [
        pl.BlockSpec((_EB, L), lambda i: (i, 0)),
        pl.BlockSpec((1, L), lambda i: (0, 0)),
    ]
    out_shape = [jax.ShapeDtypeStruct((EP, L), _f32),
                 jax.ShapeDtypeStruct((1, L), _f32)]
    args = [el, es, s_gath, g, wske, wsks, bsk, w1c, w1d, b1, w2, b2]
    if with_gate:
        in_specs += [_full((L, LA)), _full((1, LA))]
        out_specs.append(pl.BlockSpec((_EB, LA), lambda i: (i, 0)))
        out_shape.append(jax.ShapeDtypeStruct((EP, LA), _f32))
        args += [gate_w, gate_b]
    return pl.pallas_call(
        body, grid=(g_steps,), in_specs=in_specs, out_specs=out_specs,
        out_shape=out_shape,
    )(*args)


def _node_step(nd, parts, cnts, g, w1a, w1b, w1c, b1, w2, b2, wx, wy,
               proj_bias=None):
    with_bias = proj_bias is not None
    xw = LA if with_bias else L

    def body(*refs):
        if with_bias:
            (nd_r, a0_r, a1_r, c0_r, c1_r, g_r, w1a_r, w1b_r, w1c_r, b1_r,
             w2_r, b2_r, wx_r, wy_r, bp_r, nn_o, nsum_o, x_o, y_o) = refs
        else:
            (nd_r, a0_r, a1_r, c0_r, c1_r, g_r, w1a_r, w1b_r, w1c_r, b1_r,
             w2_r, b2_r, wx_r, wy_r, nn_o, nsum_o, x_o, y_o) = refs
        cnt = c0_r[...][:, :1] + c1_r[...][:, :1]
        scale = 1.0 / jnp.maximum(cnt, 1.0)
        agg = (a0_r[...] + a1_r[...]) * scale
        pre = (_dot(nd_r[...], w1a_r[...]) + _dot(agg, w1b_r[...])
               + _dot(g_r[...], w1c_r[...]) + b1_r[...])
        nn = nd_r[...] + _dot(jnp.maximum(pre, 0.0), w2_r[...]) + b2_r[...]
        nn_o[...] = nn

        @pl.when(pl.program_id(0) == 0)
        def _():
            nsum_o[...] = jnp.zeros_like(nsum_o)

        nsum_o[...] += jnp.sum(nn, axis=0, keepdims=True)
        if with_bias:
            x_o[...] = _dot(nn, wx_r[...]) + bp_r[...]
        else:
            x_o[...] = _dot(nn, wx_r[...])
        y_o[...] = _dot(nn, wy_r[...])

    g_steps = N // _NB
    off = N // _NB
    in_specs = [
        pl.BlockSpec((_NB, L), lambda i: (i, 0)),
        pl.BlockSpec((_NB, L), lambda i: (i, 0)),
        pl.BlockSpec((_NB, L), lambda i, o=off: (i + o, 0)),
        pl.BlockSpec((_NB, 16), lambda i: (i, 0)),
        pl.BlockSpec((_NB, 16), lambda i, o=off: (i + o, 0)),
        _full((1, L)), _full((L, L)), _full((L, L)), _full((L, L)),
        _full((1, L)), _full((L, L)), _full((1, L)),
        _full((L, xw)), _full((L, L)),
    ]
    args = [nd, parts, parts, cnts, cnts, g, w1a, w1b, w1c, b1, w2, b2, wx, wy]
    if with_bias:
        in_specs.append(_full((1, LA)))
        args.append(proj_bias)
    return pl.pallas_call(
        body,
        grid=(g_steps,),
        in_specs=in_specs,
        out_specs=[
            pl.BlockSpec((_NB, L), lambda i: (i, 0)),
            pl.BlockSpec((1, L), lambda i: (0, 0)),
            pl.BlockSpec((_NB, xw), lambda i: (i, 0)),
            pl.BlockSpec((_NB, L), lambda i: (i, 0)),
        ],
        out_shape=[
            jax.ShapeDtypeStruct((N, L), _f32),
            jax.ShapeDtypeStruct((1, L), _f32),
            jax.ShapeDtypeStruct((N, xw), _f32),
            jax.ShapeDtypeStruct((N, L), _f32),
        ],
    )(*args)


def _global_step(g, nsum, esum, wga, wgb, wgc, b1, w2, b2):
    def body(g_r, ns_r, es_r, wga_r, wgb_r, wgc_r, b1_r, w2_r, b2_r, o_r):
        nmean = ns_r[...] * (1.0 / N)
        emean = es_r[...] * (1.0 / E)
        pre = (_dot(g_r[...], wga_r[...]) + _dot(nmean, wgb_r[...])
               + _dot(emean, wgc_r[...]) + b1_r[...])
        o_r[...] = g_r[...] + _dot(jnp.maximum(pre, 0.0), w2_r[...]) + b2_r[...]

    return pl.pallas_call(
        body,
        grid=(1,),
        in_specs=[_full((1, L))] * 3 + [_full((L, L))] * 3
        + [_full((1, L)), _full((L, L)), _full((1, L))],
        out_specs=_full((1, L)),
        out_shape=jax.ShapeDtypeStruct((1, L), _f32),
    )(g, nsum, esum, wga, wgb, wgc, b1, w2, b2)


def _apply_input(cache, v2, wa, wv, b1, w2, b2, wm, bm):
    def body(c_r, v_r, wa_r, wv_r, b1_r, w2_r, b2_r, wm_r, bm_r, h_o, hm_o):
        pre = _dot(c_r[...], wa_r[...]) + _dot(v_r[...], wv_r[...]) + b1_r[...]
        h = jnp.maximum(pre, 0.0)
        h = _dot(h, w2_r[...]) + b2_r[...]
        h_o[...] = h
        hm_o[...] = _dot(h, wm_r[...]) + bm_r[...]

    g_steps = N // _NB
    return pl.pallas_call(
        body,
        grid=(g_steps,),
        in_specs=[
            pl.BlockSpec((_NB, LA), lambda i: (i, 0)),
            pl.BlockSpec((_NB, 2), lambda i: (i, 0)),
            _full((LA, LA)), _full((2, LA)), _full((1, LA)),
            _full((LA, LA)), _full((1, LA)),
            _full((LA, LA)), _full((1, LA)),
        ],
        out_specs=[
            pl.BlockSpec((_NB, LA), lambda i: (i, 0)),
            pl.BlockSpec((_NB, LA), lambda i: (i, 0)),
        ],
        out_shape=[jax.ShapeDtypeStruct((N, LA), _f32)] * 2,
    )(cache, v2, wa, wv, b1, w2, b2, wm, bm)


def _apply_final(h, parts, cnts, v2, wha, whb, b1, w2, b2, wo, bo):
    def body(h_r, a0_r, a1_r, c0_r, c1_r, v_r, wha_r, whb_r, b1_r, w2_r,
             b2_r, wo_r, bo_r, o_r):
        cnt = c0_r[...][:, :1] + c1_r[...][:, :1]
        scale = 1.0 / jnp.maximum(cnt, 1.0)
        agg = (a0_r[...] + a1_r[...]) * scale
        pre = _dot(h_r[...], wha_r[...]) + _dot(agg, whb_r[...]) + b1_r[...]
        h2 = h_r[...] + _dot(jnp.maximum(pre, 0.0), w2_r[...]) + b2_r[...]
        o_r[...] = v_r[...] + _dot(h2, wo_r[...]) + bo_r[...]

    g_steps = N // _NB
    off = N // _NB
    return pl.pallas_call(
        body,
        grid=(g_steps,),
        in_specs=[
            pl.BlockSpec((_NB, LA), lambda i: (i, 0)),
            pl.BlockSpec((_NB, LA), lambda i: (i, 0)),
            pl.BlockSpec((_NB, LA), lambda i, o=off: (i + o, 0)),
            pl.BlockSpec((_NB, 16), lambda i: (i, 0)),
            pl.BlockSpec((_NB, 16), lambda i, o=off: (i + o, 0)),
            pl.BlockSpec((_NB, 2), lambda i: (i, 0)),
            _full((LA, LA)), _full((LA, LA)), _full((1, LA)),
            _full((LA, LA)), _full((1, LA)),
            _full((LA, 2)), _full((1, 2)),
        ],
        out_specs=pl.BlockSpec((_NB, 2), lambda i: (i, 0)),
        out_shape=jax.ShapeDtypeStruct((N, 2), _f32),
    )(h, parts, parts, cnts, cnts, v2, wha, whb, b1, w2, b2, wo, bo)


def _r(b):
    return b.reshape(1, -1)


def _forward(x, edge_index, edge_attr, global_features, v_re, v_im, params):
    prm = params
    pad = jnp.zeros((EP - E,), jnp.int32)
    row2 = jnp.concatenate([edge_index[0], pad]).reshape(NCHP, CH)
    col2 = jnp.concatenate([edge_index[1], pad]).reshape(NCHP, CH)

    proc = prm['processor']
    skips = prm['edge_skip']
    e1 = [p['edge_mlp'][0][0] for p in proc]
    w1a = [w[0:L] for w in e1]
    w1b = [w[L:2 * L] for w in e1]
    w1c = [w[2 * L:3 * L] for w in e1]
    w1d = [w[3 * L:4 * L] for w in e1]

    (ne_w0, ne_b0), (ne_w1, ne_b1) = prm['node_enc']
    (ee_w0, ee_b0), (ee_w1, ee_b1) = prm['edge_enc']
    (ge_w0, ge_b0), (ge_w1, ge_b1) = prm['global_enc']

    node, p_cur, q_cur = _node_enc(x, ne_w0, _r(ne_b0), ne_w1, _r(ne_b1),
                                   w1a[0], w1b[0])
    e_saved = _edge_enc(edge_attr, ee_w0, _r(ee_b0), ee_w1, _r(ee_b1))
    g = _global_enc(global_features, ge_w0, _r(ge_b0), ge_w1, _r(ge_b1))

    cnts = _sc_counts(row2)

    edge = e_saved
    gate = None
    cache = None
    (gate_w, gate_b) = prm['edge_gate']
    (proj_w, proj_b) = prm['node_proj']
    aux = {}
    for i in range(MP_STEPS):
        sk_w, sk_b = skips[i]
        pe = proc[i]['edge_mlp']
        pn = proc[i]['node_mlp']
        pg = proc[i]['global_mlp']
        b1 = pe[0][1]
        w2, b2 = pe[1]
        sgath = _sc_gather_sum(p_cur, q_cur, row2, col2)
        last = i == MP_STEPS - 1
        eres = _edge_step(edge, e_saved, sgath, g,
                          sk_w[0:L], sk_w[L:2 * L], _r(sk_b),
                          w1c[i], w1d[i], _r(b1), w2, _r(b2),
                          gate_w=gate_w if last else None,
                          gate_b=_r(gate_b) if last else None)
        if last:
            edge, esum, gate = eres
        else:
            edge, esum = eres
        parts = _sc_scatter(edge, row2)
        n1 = pn[0][0]
        if last:
            node, nsum, cache, _dummy = _node_step(
                node, parts, cnts, g,
                n1[0:L], n1[L:2 * L], n1[2 * L:3 * L], _r(pn[0][1]),
                pn[1][0], _r(pn[1][1]), proj_w, pn[1][0],
                proj_bias=_r(proj_b))
        else:
            node, nsum, p_cur, q_cur = _node_step(
                node, parts, cnts, g,
                n1[0:L], n1[L:2 * L], n1[2 * L:3 * L], _r(pn[0][1]),
                pn[1][0], _r(pn[1][1]), w1a[i + 1], w1b[i + 1])
        g1 = pg[0][0]
        g = _global_step(g, nsum, esum,
                         g1[0:L], g1[L:2 * L], g1[2 * L:3 * L],
                         _r(pg[0][1]), pg[1][0], _r(pg[1][1]))

    aux['node'] = node
    aux['edge'] = edge
    aux['g'] = g
    aux['gate'] = gate
    aux['cache'] = cache

    v2 = jnp.stack([v_re, v_im], axis=1)
    ai = prm['apply_input']
    a1w = ai[0][0]
    mp0 = prm['apply_mp'][0]
    (wm, bm) = mp0['msg_proj']
    h, hm = _apply_input(cache, v2, a1w[0:LA], a1w[LA:LA + 2], _r(ai[0][1]),
                         ai[1][0], _r(ai[1][1]), wm, _r(bm))
    aux['h'] = h
    aux['hm'] = hm

    aparts = _sc_apply(hm, col2, gate, row2)
    aux['aparts'] = aparts

    u1 = mp0['upd'][0][0]
    (wo, bo) = prm['apply_output']
    out2 = _apply_final(h, aparts, cnts, v2,
                        u1[0:LA], u1[LA:2 * LA], _r(mp0['upd'][0][1]),
                        mp0['upd'][1][0], _r(mp0['upd'][1][1]), wo, _r(bo))
    aux['out2'] = out2
    return out2[:, 0], out2[:, 1], aux


def kernel(x, edge_index, edge_attr, global_features, v_re, v_im, params):
    out_re, out_im, _ = _forward(x, edge_index, edge_attr, global_features,
                                 v_re, v_im, params)
    return out_re, out_im
